# baseline (speedup 1.0000x reference)
import functools

import jax
import jax.numpy as jnp
from jax import lax
from jax.experimental import pallas as pl
from jax.experimental.pallas import tpu as pltpu
from jax.experimental.pallas import tpu_sc as plsc

N_NODES = 10000
N_EDGES = 320000
D = 128

NC = 2
NS = 16
NW = NC * NS
CHUNK = 80
NCHUNK = 128
EDGES_PER_W = NCHUNK * CHUNK
N_ACC = 10240
ROWS_PER_TILE = N_ACC // NS


def _mm_body(x_ref, w_ref, n_ref, h_ref):
    h_ref[...] = jnp.dot(x_ref[...], w_ref[...],
                         preferred_element_type=jnp.float32) * n_ref[...]


def _matmul_norm(x, W, norm):
    M_BLK = 1000
    return pl.pallas_call(
        _mm_body,
        grid=(N_NODES // M_BLK,),
        in_specs=[
            pl.BlockSpec((M_BLK, D), lambda i: (i, 0)),
            pl.BlockSpec((D, D), lambda i: (0, 0)),
            pl.BlockSpec((M_BLK, 1), lambda i: (i, 0)),
        ],
        out_specs=pl.BlockSpec((M_BLK, D), lambda i: (i, 0)),
        out_shape=jax.ShapeDtypeStruct((N_NODES, D), jnp.float32),
    )(x, W, norm.reshape(N_NODES, 1))


_MESH = plsc.VectorSubcoreMesh(core_axis_name="c", subcore_axis_name="s")


@functools.partial(
    pl.kernel,
    out_type=jax.ShapeDtypeStruct((NC, N_ACC, D), jnp.float32),
    mesh=_MESH,
    scratch_types=[
        pltpu.VMEM_SHARED((N_ACC, D), jnp.float32),
        pltpu.VMEM((CHUNK,), jnp.int32),
        pltpu.VMEM((CHUNK,), jnp.int32),
        pltpu.VMEM((CHUNK,), jnp.int32),
        pltpu.VMEM((CHUNK,), jnp.int32),
        pltpu.VMEM((CHUNK,), jnp.int32),
        pltpu.VMEM((CHUNK,), jnp.int32),
        pltpu.VMEM((CHUNK,), jnp.int32),
        pltpu.VMEM((CHUNK,), jnp.int32),
        pltpu.VMEM((CHUNK, D), jnp.float32),
        pltpu.VMEM((CHUNK, D), jnp.float32),
        pltpu.VMEM((CHUNK, D), jnp.float32),
        pltpu.VMEM((CHUNK, D), jnp.float32),
        pltpu.SemaphoreType.DMA,
        pltpu.SemaphoreType.DMA,
        pltpu.SemaphoreType.DMA,
        pltpu.SemaphoreType.DMA,
    ],
)
def _edge_scatter(h_hbm, src_hbm, dst_hbm, zero_hbm, out_hbm,
                  acc, src0, src1, src2, src3, dst0, dst1, dst2, dst3,
                  rows0, rows1, rows2, rows3, sem0, sem1, sem2, sem3):
    cid = lax.axis_index("c")
    sid = lax.axis_index("s")
    wid = sid * NC + cid

    row0 = sid * ROWS_PER_TILE
    pltpu.sync_copy(zero_hbm.at[pl.ds(row0, ROWS_PER_TILE)],
                    acc.at[pl.ds(row0, ROWS_PER_TILE)])
    plsc.subcore_barrier()

    cbase = wid * NCHUNK

    rows = (rows0, rows1, rows2, rows3)
    srcs = (src0, src1, src2, src3)
    dsts = (dst0, dst1, dst2, dst3)
    sems = (sem0, sem1, sem2, sem3)
    for p in range(4):
        pltpu.sync_copy(src_hbm.at[cbase + p], srcs[p])
        pltpu.sync_copy(dst_hbm.at[cbase + p], dsts[p])
        pltpu.async_copy(h_hbm.at[srcs[p]], rows[p], sems[p])

    def body(i, carry):
        j = 4 * i
        for p in range(4):
            pltpu.make_async_copy(h_hbm.at[srcs[p]], rows[p], sems[p]).wait()
            pltpu.sync_copy(src_hbm.at[cbase + j + 4 + p], srcs[p])
            pltpu.sync_copy(dst_hbm.at[cbase + j + 4 + p], dsts[p])
            pltpu.async_copy(h_hbm.at[srcs[p]], rows[p], sems[p])
        return carry

    lax.fori_loop(0, NCHUNK // 4, body, 0)
    for p in range(4):
        pltpu.make_async_copy(h_hbm.at[srcs[p]], rows[p], sems[p]).wait()

    plsc.subcore_barrier()

    @pl.when(cid == 0)
    def _():
        pltpu.sync_copy(acc.at[pl.ds(row0, ROWS_PER_TILE)],
                        out_hbm.at[0].at[pl.ds(row0, ROWS_PER_TILE)])

    @pl.when(cid == 1)
    def _():
        pltpu.sync_copy(acc.at[pl.ds(row0, ROWS_PER_TILE)],
                        out_hbm.at[1].at[pl.ds(row0, ROWS_PER_TILE)])


def _post_body(p_ref, n_ref, b_ref, o_ref):
    s = p_ref[0] + p_ref[1]
    o_ref[...] = jnp.maximum(s * n_ref[...] + b_ref[...], 0.0)


def _postprocess(partials, norm, b):
    M_BLK = 1000
    return pl.pallas_call(
        _post_body,
        grid=(N_NODES // M_BLK,),
        in_specs=[
            pl.BlockSpec((NC, M_BLK, D), lambda i: (0, i, 0)),
            pl.BlockSpec((M_BLK, 1), lambda i: (i, 0)),
            pl.BlockSpec((1, D), lambda i: (0, 0)),
        ],
        out_specs=pl.BlockSpec((M_BLK, D), lambda i: (i, 0)),
        out_shape=jax.ShapeDtypeStruct((N_NODES, D), jnp.float32),
    )(partials, norm.reshape(N_NODES, 1), b.reshape(1, D))


def kernel(x, edge_index, norm, W, b):
    h = _matmul_norm(x, W, norm)
    ei = edge_index.astype(jnp.int32)
    pad_n = NW * EDGES_PER_W - N_EDGES
    src = jnp.concatenate([ei[0], jnp.zeros((pad_n,), jnp.int32)])
    dst = jnp.concatenate(
        [ei[1],
         N_NODES + (jnp.arange(pad_n, dtype=jnp.int32) % (N_ACC - N_NODES))])
    dummy = jnp.zeros((4 * CHUNK,), jnp.int32)
    src_r = jnp.concatenate([src, dummy]).reshape(NW * NCHUNK + 4, CHUNK)
    dst_r = jnp.concatenate([dst, dummy]).reshape(NW * NCHUNK + 4, CHUNK)
    zeros = jnp.zeros((N_ACC, D), dtype=jnp.float32)
    partials = _edge_scatter(h, src_r, dst_r, zeros)
    return _postprocess(partials, norm, b)


# speedup vs baseline: 1.2756x; 1.2756x over previous
import functools

import jax
import jax.numpy as jnp
from jax import lax
from jax.experimental import pallas as pl
from jax.experimental.pallas import tpu as pltpu
from jax.experimental.pallas import tpu_sc as plsc

N_NODES = 10000
N_EDGES = 320000
D = 128
DH = D // 2

NC = 2
NS = 16
CHUNK = 80
NCHUNK = 252
EDGES_PER_T = NCHUNK * CHUNK
N_ACC = 10240
ROWS_PER_TILE = N_ACC // NS


def _mm_body(x_ref, w_ref, n_ref, h0_ref, h1_ref):
    h = jnp.dot(x_ref[...], w_ref[...],
                preferred_element_type=jnp.float32) * n_ref[...]
    h0_ref[...] = h[:, :DH]
    h1_ref[...] = h[:, DH:]


def _matmul_norm(x_p, W, norm_p):
    M_BLK = 640
    return pl.pallas_call(
        _mm_body,
        grid=(N_ACC // M_BLK,),
        in_specs=[
            pl.BlockSpec((M_BLK, D), lambda i: (i, 0)),
            pl.BlockSpec((D, D), lambda i: (0, 0)),
            pl.BlockSpec((M_BLK, 1), lambda i: (i, 0)),
        ],
        out_specs=[
            pl.BlockSpec((M_BLK, DH), lambda i: (i, 0)),
            pl.BlockSpec((M_BLK, DH), lambda i: (i, 0)),
        ],
        out_shape=[
            jax.ShapeDtypeStruct((N_ACC, DH), jnp.float32),
            jax.ShapeDtypeStruct((N_ACC, DH), jnp.float32),
        ],
    )(x_p, W, norm_p.reshape(N_ACC, 1))


_MESH = plsc.VectorSubcoreMesh(core_axis_name="c", subcore_axis_name="s")


@functools.partial(
    pl.kernel,
    out_type=jax.ShapeDtypeStruct((NC, N_ACC, DH), jnp.float32),
    mesh=_MESH,
    scratch_types=[
        pltpu.VMEM_SHARED((N_ACC, DH), jnp.float32),
        pltpu.VMEM_SHARED((N_ACC, DH), jnp.float32),
        pltpu.VMEM((CHUNK,), jnp.int32),
        pltpu.VMEM((CHUNK,), jnp.int32),
        pltpu.VMEM((CHUNK,), jnp.int32),
        pltpu.VMEM((CHUNK,), jnp.int32),
        pltpu.VMEM((CHUNK, DH), jnp.float32),
        pltpu.VMEM((CHUNK, DH), jnp.float32),
        pltpu.SemaphoreType.DMA,
        pltpu.SemaphoreType.DMA,
    ],
)
def _edge_scatter(h0_hbm, h1_hbm, src_hbm, dst_hbm, zero_hbm, out_hbm,
                  hres, acc, src0, src1, dst0, dst1, rows0, rows1,
                  sem0, sem1):
    cid = lax.axis_index("c")
    sid = lax.axis_index("s")

    row0 = sid * ROWS_PER_TILE

    @pl.when(cid == 0)
    def _():
        pltpu.sync_copy(h0_hbm.at[pl.ds(row0, ROWS_PER_TILE)],
                        hres.at[pl.ds(row0, ROWS_PER_TILE)])

    @pl.when(cid == 1)
    def _():
        pltpu.sync_copy(h1_hbm.at[pl.ds(row0, ROWS_PER_TILE)],
                        hres.at[pl.ds(row0, ROWS_PER_TILE)])

    pltpu.sync_copy(zero_hbm.at[pl.ds(row0, ROWS_PER_TILE)],
                    acc.at[pl.ds(row0, ROWS_PER_TILE)])
    plsc.subcore_barrier()

    cbase = sid * NCHUNK

    pltpu.sync_copy(src_hbm.at[cbase], src0)
    pltpu.sync_copy(dst_hbm.at[cbase], dst0)
    pltpu.async_copy(hres.at[src0], rows0, sem0)
    pltpu.sync_copy(src_hbm.at[cbase + 1], src1)
    pltpu.sync_copy(dst_hbm.at[cbase + 1], dst1)
    pltpu.async_copy(hres.at[src1], rows1, sem1)

    def body(i, carry):
        j = 2 * i
        pltpu.make_async_copy(hres.at[src0], rows0, sem0).wait()
        pltpu.sync_copy(rows0, acc.at[dst0], add=True)
        pltpu.sync_copy(src_hbm.at[cbase + j + 2], src0)
        pltpu.sync_copy(dst_hbm.at[cbase + j + 2], dst0)
        pltpu.async_copy(hres.at[src0], rows0, sem0)
        pltpu.make_async_copy(hres.at[src1], rows1, sem1).wait()
        pltpu.sync_copy(rows1, acc.at[dst1], add=True)
        pltpu.sync_copy(src_hbm.at[cbase + j + 3], src1)
        pltpu.sync_copy(dst_hbm.at[cbase + j + 3], dst1)
        pltpu.async_copy(hres.at[src1], rows1, sem1)
        return carry

    lax.fori_loop(0, NCHUNK // 2, body, 0)
    pltpu.make_async_copy(hres.at[src0], rows0, sem0).wait()
    pltpu.make_async_copy(hres.at[src1], rows1, sem1).wait()

    plsc.subcore_barrier()

    @pl.when(cid == 0)
    def _():
        pltpu.sync_copy(acc.at[pl.ds(row0, ROWS_PER_TILE)],
                        out_hbm.at[0].at[pl.ds(row0, ROWS_PER_TILE)])

    @pl.when(cid == 1)
    def _():
        pltpu.sync_copy(acc.at[pl.ds(row0, ROWS_PER_TILE)],
                        out_hbm.at[1].at[pl.ds(row0, ROWS_PER_TILE)])


def _post_body(p0_ref, p1_ref, n_ref, b_ref, o_ref):
    s = jnp.concatenate([p0_ref[0], p1_ref[0]], axis=1)
    o_ref[...] = jnp.maximum(s * n_ref[...] + b_ref[...], 0.0)


def _postprocess(partials, norm, b):
    M_BLK = 1000
    return pl.pallas_call(
        _post_body,
        grid=(N_NODES // M_BLK,),
        in_specs=[
            pl.BlockSpec((1, M_BLK, DH), lambda i: (0, i, 0)),
            pl.BlockSpec((1, M_BLK, DH), lambda i: (1, i, 0)),
            pl.BlockSpec((M_BLK, 1), lambda i: (i, 0)),
            pl.BlockSpec((1, D), lambda i: (0, 0)),
        ],
        out_specs=pl.BlockSpec((M_BLK, D), lambda i: (i, 0)),
        out_shape=jax.ShapeDtypeStruct((N_NODES, D), jnp.float32),
    )(partials, partials, norm.reshape(N_NODES, 1), b.reshape(1, D))


def kernel(x, edge_index, norm, W, b):
    x_p = jnp.concatenate(
        [x, jnp.zeros((N_ACC - N_NODES, D), jnp.float32)], axis=0)
    norm_p = jnp.concatenate(
        [norm, jnp.zeros((N_ACC - N_NODES,), jnp.float32)])
    h0, h1 = _matmul_norm(x_p, W, norm_p)
    ei = edge_index.astype(jnp.int32)
    pad_n = NS * EDGES_PER_T - N_EDGES
    src = jnp.concatenate([ei[0], jnp.zeros((pad_n,), jnp.int32)])
    dst = jnp.concatenate(
        [ei[1],
         N_NODES + (jnp.arange(pad_n, dtype=jnp.int32) % (N_ACC - N_NODES))])
    dummy = jnp.zeros((2 * CHUNK,), jnp.int32)
    src_r = jnp.concatenate([src, dummy]).reshape(NS * NCHUNK + 2, CHUNK)
    dst_r = jnp.concatenate([dst, dummy]).reshape(NS * NCHUNK + 2, CHUNK)
    zeros = jnp.zeros((N_ACC, DH), dtype=jnp.float32)
    partials = _edge_scatter(h0, h1, src_r, dst_r, zeros)
    return _postprocess(partials, norm, b)
